# Initial kernel scaffold; baseline (speedup 1.0000x reference)
import jax
import jax.numpy as jnp
from jax.experimental import pallas as pl


def kernel(input_stream, permutation, W1a, b1a, W2a, b2a, W1b, b1b, W2b, b2b, noise_sys, noise_par1, noise_par2, possible_inputs, next_states, prev_states):
    raise NotImplementedError("write your pallas kernel here")


# speedup vs baseline: 4.5210x; 4.5210x over previous
import functools

import jax
import jax.numpy as jnp
from jax import lax
from jax.experimental import pallas as pl
from jax.experimental.pallas import tpu as pltpu
from jax.experimental.pallas import tpu_sc as plsc

B, L, WIN, H = 64, 4096, 5, 64
NUM_ST, NUM_IN = 16, 2
SIGMA = 0.5
NTAB = NUM_ST * NUM_IN
D = 2 * B
NW = 32
ROWS_PER_W = L // NW


@functools.lru_cache(maxsize=None)
def _make_sc_gather():
    mesh = plsc.VectorSubcoreMesh(core_axis_name="c", subcore_axis_name="s")

    @functools.partial(
        pl.kernel,
        out_type=jax.ShapeDtypeStruct((L, D), jnp.float32),
        mesh=mesh,
        scratch_types=[
            pltpu.VMEM((ROWS_PER_W,), jnp.int32),
            pltpu.VMEM((ROWS_PER_W, D), jnp.float32),
            pltpu.SemaphoreType.DMA,
        ],
    )
    def sc_gather(table_hbm, idx_hbm, out_hbm, idx_v, rows_v, sem):
        wid = lax.axis_index("s") * 2 + lax.axis_index("c")
        base = wid * ROWS_PER_W
        pltpu.sync_copy(idx_hbm.at[pl.ds(base, ROWS_PER_W)], idx_v)
        pltpu.async_copy(table_hbm.at[idx_v], rows_v, sem).wait()
        pltpu.sync_copy(rows_v, out_hbm.at[pl.ds(base, ROWS_PER_W)])

    return sc_gather


def _tc_body(bits, bp, nsp, ns, n1, n2, pi, w1a, b1a, w2a, b2a, w1b, b1b,
             w2b, b2b, o_sys, o_par1, o_isys, o_par2, o_c1, o_c2):
    bits_i = bits[...]
    xb = 2.0 * bits_i.astype(jnp.float32) - 1.0
    wb = 2.0 * pi[...] - 1.0

    def table(w1, b1, w2, b2):
        h = jnp.tanh(jnp.dot(wb, w1[...],
                             preferred_element_type=jnp.float32) + b1[...])
        t = jnp.tanh(jnp.dot(h, w2[...],
                             preferred_element_type=jnp.float32) + b2[...])
        return t[:, 0]

    ta = table(w1a, b1a, w2a, b2a)
    tb = table(w1b, b1b, w2b, b2b)

    def widx(b):
        acc = b
        for k in range(1, WIN):
            sh = jnp.concatenate(
                [jnp.zeros((B, k), jnp.int32), b[:, : L - k]], axis=1)
            acc = acc + (1 << k) * sh
        return acc

    def lookup(idx, t):
        acc = jnp.zeros((B, L), jnp.float32)
        for n in range(NTAB):
            acc = acc + jnp.where(idx == n, t[n], 0.0)
        return acc

    idx_a = widx(bits_i)
    pa = lookup(idx_a, ta)
    m1 = jnp.mean(pa)
    s1 = jnp.sqrt(jnp.mean((pa - m1) ** 2))

    bpf = bp[...]
    idx_b = widx(bpf.astype(jnp.int32))
    pb = lookup(idx_b, tb)
    m2 = jnp.mean(pb)
    s2 = jnp.sqrt(jnp.mean((pb - m2) ** 2))

    o_sys[...] = xb + SIGMA * ns[...]
    o_par1[...] = (pa - m1) / s1 + SIGMA * n1[...]
    o_par2[...] = (pb - m2) / s2 + SIGMA * n2[...]
    o_isys[...] = (2.0 * bpf - 1.0) + SIGMA * nsp[...]
    o_c1[...] = jnp.concatenate(
        [wb[:, WIN - 1:WIN], ((ta - m1) / s1)[:, None]], axis=1)
    o_c2[...] = jnp.concatenate(
        [wb[:, WIN - 1:WIN], ((tb - m2) / s2)[:, None]], axis=1)


def _tc_call(bits, bp, nsp, ns, n1, n2, pi, *weights):
    return pl.pallas_call(
        _tc_body,
        out_shape=[
            jax.ShapeDtypeStruct((B, L), jnp.float32),
            jax.ShapeDtypeStruct((B, L), jnp.float32),
            jax.ShapeDtypeStruct((B, L), jnp.float32),
            jax.ShapeDtypeStruct((B, L), jnp.float32),
            jax.ShapeDtypeStruct((NTAB, 2), jnp.float32),
            jax.ShapeDtypeStruct((NTAB, 2), jnp.float32),
        ],
    )(bits, bp, nsp, ns, n1, n2, pi, *weights)


def kernel(input_stream, permutation, W1a, b1a, W2a, b2a, W1b, b1b, W2b, b2b,
           noise_sys, noise_par1, noise_par2, possible_inputs, next_states,
           prev_states):
    bits = input_stream.astype(jnp.int32)
    ns = noise_sys[:, :, 0]
    packed = jnp.concatenate(
        [input_stream.astype(jnp.float32).T, ns.T], axis=1)
    g = _make_sc_gather()(packed, permutation.astype(jnp.int32))
    bp = g[:, :B].T
    nsp = g[:, B:].T
    o_sys, o_par1, o_isys, o_par2, c1, c2 = _tc_call(
        bits, bp, nsp, ns, noise_par1[:, :, 0], noise_par2[:, :, 0],
        possible_inputs,
        W1a, b1a.reshape(1, H), W2a, b2a.reshape(1, 1),
        W1b, b1b.reshape(1, H), W2b, b2b.reshape(1, 1))
    expand = lambda x: x[:, :, None]
    return (expand(o_sys), expand(o_par1), expand(o_isys), expand(o_par2),
            c1.reshape(NUM_ST, NUM_IN, 2), c2.reshape(NUM_ST, NUM_IN, 2))
